# baseline (speedup 1.0000x reference)
import functools

import jax
import jax.numpy as jnp
from jax import lax
from jax.experimental import pallas as pl
from jax.experimental.pallas import tpu as pltpu
from jax.experimental.pallas import tpu_sc as plsc

SEQ_LEN = 8192
DIM = 1024
NUM_CORES = 2
NUM_SUBCORES = 16
NUM_WORKERS = NUM_CORES * NUM_SUBCORES
ROWS_PER_WORKER = SEQ_LEN // NUM_WORKERS

_mesh = plsc.VectorSubcoreMesh(core_axis_name="c", subcore_axis_name="s")


@functools.partial(
    pl.kernel,
    mesh=_mesh,
    out_type=jax.ShapeDtypeStruct((SEQ_LEN, DIM), jnp.float32),
)
def _pos_embed_lookup(table_hbm, out_hbm):
    wid = lax.axis_index("s") * NUM_CORES + lax.axis_index("c")
    base = wid * ROWS_PER_WORKER
    pltpu.sync_copy(
        table_hbm.at[pl.ds(base, ROWS_PER_WORKER)],
        out_hbm.at[pl.ds(base, ROWS_PER_WORKER)],
    )


def kernel(x, emb_weight):
    del x
    return _pos_embed_lookup(emb_weight)


# speedup vs baseline: 23.7484x; 23.7484x over previous
import functools

import jax
import jax.numpy as jnp
from jax import lax
from jax.experimental import pallas as pl
from jax.experimental.pallas import tpu as pltpu
from jax.experimental.pallas import tpu_sc as plsc

SEQ_LEN = 8192
DIM = 1024
NUM_CORES = 2
NUM_SUBCORES = 16
NUM_WORKERS = NUM_CORES * NUM_SUBCORES
ROWS_PER_WORKER = SEQ_LEN // NUM_WORKERS
CHUNK = 64
NCHUNK = ROWS_PER_WORKER // CHUNK

_mesh = plsc.VectorSubcoreMesh(core_axis_name="c", subcore_axis_name="s")


@functools.partial(
    pl.kernel,
    mesh=_mesh,
    out_type=jax.ShapeDtypeStruct((SEQ_LEN, DIM), jnp.float32),
    scratch_types=[pltpu.VMEM((CHUNK, DIM), jnp.float32)],
)
def _pos_embed_lookup(table_hbm, out_hbm, buf_v):
    wid = lax.axis_index("s") * NUM_CORES + lax.axis_index("c")
    base = wid * ROWS_PER_WORKER

    def body(i, carry):
        r0 = base + i * CHUNK
        pltpu.sync_copy(table_hbm.at[pl.ds(r0, CHUNK)], buf_v)
        pltpu.sync_copy(buf_v, out_hbm.at[pl.ds(r0, CHUNK)])
        return carry

    lax.fori_loop(0, NCHUNK, body, 0)


def kernel(x, emb_weight):
    del x
    return _pos_embed_lookup(emb_weight)
